# Initial kernel scaffold; baseline (speedup 1.0000x reference)
import jax
import jax.numpy as jnp
from jax.experimental import pallas as pl


def kernel(x, table, W, b):
    raise NotImplementedError("write your pallas kernel here")


# speedup vs baseline: 17.7279x; 17.7279x over previous
import functools

import jax
import jax.numpy as jnp
from jax import lax
from jax.experimental import pallas as pl
from jax.experimental.pallas import tpu as pltpu
from jax.experimental.pallas import tpu_sc as plsc

_VOCAB = 100000
_DIM = 128
_BATCH = 4096
_SEQ = 200

_VR = 800
_VC = 125
_ROWS_BLK = 8

_NC = 2
_NS = 16
_NW = _NC * _NS
_ROWS_PER_TILE = _BATCH // _NW
_LANES = 16


def _scores_body(t_ref, w_ref, b_ref, o_ref):
    i = pl.program_id(0)
    t = t_ref[...]
    w = w_ref[...]
    s = jnp.sum(t * w, axis=-1) * (1.0 / _SEQ)
    r = lax.broadcasted_iota(jnp.int32, (_ROWS_BLK, _VC), 0)
    c = lax.broadcasted_iota(jnp.int32, (_ROWS_BLK, _VC), 1)
    is_pad = (i == 0) & (r == 0) & (c == 0)
    o_ref[...] = jnp.where(is_pad, 0.0, s) + b_ref[0] * (1.0 / _SEQ)


_scores_call = pl.pallas_call(
    _scores_body,
    grid=(_VR // _ROWS_BLK,),
    in_specs=[
        pl.BlockSpec((_ROWS_BLK, _VC, _DIM), lambda i: (i, 0, 0)),
        pl.BlockSpec((1, _DIM), lambda i: (0, 0)),
        pl.BlockSpec(memory_space=pltpu.SMEM),
    ],
    out_specs=pl.BlockSpec((_ROWS_BLK, _VC), lambda i: (i, 0)),
    out_shape=jax.ShapeDtypeStruct((_VR, _VC), jnp.float32),
)


@functools.partial(
    pl.kernel,
    out_type=jax.ShapeDtypeStruct((_BATCH,), jnp.float32),
    mesh=plsc.VectorSubcoreMesh(core_axis_name="c", subcore_axis_name="s"),
    compiler_params=pltpu.CompilerParams(needs_layout_passes=False),
    scratch_types=[
        pltpu.VMEM((_VOCAB,), jnp.float32),
        pltpu.VMEM((_ROWS_PER_TILE * _SEQ,), jnp.int32),
        pltpu.VMEM((_ROWS_PER_TILE,), jnp.float32),
    ],
)
def _sc_pool(scores_hbm, x_hbm, out_hbm, scores_v, x_v, out_v):
    wid = lax.axis_index("s") * _NC + lax.axis_index("c")
    base = wid * _ROWS_PER_TILE
    pltpu.sync_copy(scores_hbm, scores_v)
    pltpu.sync_copy(x_hbm.at[pl.ds(base * _SEQ, _ROWS_PER_TILE * _SEQ)], x_v)
    lane = lax.iota(jnp.int32, _LANES)
    for g in range(_ROWS_PER_TILE // _LANES):
        base_vec = (lane + g * _LANES) * _SEQ

        def body(j, acc, base_vec=base_vec):
            idx = plsc.load_gather(x_v, [base_vec + j])
            return acc + plsc.load_gather(scores_v, [idx])

        acc = lax.fori_loop(0, _SEQ, body, jnp.zeros((_LANES,), jnp.float32))
        out_v[pl.ds(g * _LANES, _LANES)] = acc
    pltpu.sync_copy(out_v, out_hbm.at[pl.ds(base, _ROWS_PER_TILE)])


def kernel(x, table, W, b):
    t3 = table.reshape(_VR, _VC, _DIM)
    scores = _scores_call(t3, W, b).reshape(_VOCAB)
    out = _sc_pool(scores, x.astype(jnp.int32).reshape(_BATCH * _SEQ))
    return out.reshape(_BATCH, 1)


# speedup vs baseline: 21.7797x; 1.2286x over previous
import functools

import jax
import jax.numpy as jnp
from jax import lax
from jax.experimental import pallas as pl
from jax.experimental.pallas import tpu as pltpu
from jax.experimental.pallas import tpu_sc as plsc

_VOCAB = 100000
_DIM = 128
_BATCH = 4096
_SEQ = 200

_ROWS_BLK = 1000
_NBLK = _VOCAB // _ROWS_BLK

_NC = 2
_NS = 16
_NW = _NC * _NS
_ROWS_PER_TILE = _BATCH // _NW
_LANES = 16
_UNROLL = 8


def _scores_body(t_ref, w_ref, b_ref, o_ref):
    i = pl.program_id(0)
    t = t_ref[...]
    w = w_ref[...]
    s = jnp.sum(t * w, axis=-1) * (1.0 / _SEQ)
    o_ref[...] = s.reshape(1, 1, _ROWS_BLK) + b_ref[0] * (1.0 / _SEQ)

    @pl.when(i == 0)
    def _():
        head = o_ref[0, 0, pl.ds(0, _DIM)]
        sel = lax.broadcasted_iota(jnp.int32, (_DIM,), 0) == 0
        o_ref[0, 0, pl.ds(0, _DIM)] = jnp.where(sel, b_ref[0] * (1.0 / _SEQ), head)


_scores_call = pl.pallas_call(
    _scores_body,
    grid=(_NBLK,),
    in_specs=[
        pl.BlockSpec((_ROWS_BLK, _DIM), lambda i: (i, 0)),
        pl.BlockSpec((1, _DIM), lambda i: (0, 0)),
        pl.BlockSpec(memory_space=pltpu.SMEM),
    ],
    out_specs=pl.BlockSpec((1, 1, _ROWS_BLK), lambda i: (i, 0, 0)),
    out_shape=jax.ShapeDtypeStruct((_NBLK, 1, _ROWS_BLK), jnp.float32),
)


@functools.partial(
    pl.kernel,
    out_type=jax.ShapeDtypeStruct((_BATCH,), jnp.float32),
    mesh=plsc.VectorSubcoreMesh(core_axis_name="c", subcore_axis_name="s"),
    compiler_params=pltpu.CompilerParams(needs_layout_passes=False),
    scratch_types=[
        pltpu.VMEM((_VOCAB,), jnp.float32),
        pltpu.VMEM((_ROWS_PER_TILE * _SEQ,), jnp.int32),
        pltpu.VMEM((_ROWS_PER_TILE,), jnp.float32),
    ],
)
def _sc_pool(scores_hbm, x_hbm, out_hbm, scores_v, x_v, out_v):
    wid = lax.axis_index("s") * _NC + lax.axis_index("c")
    base = wid * _ROWS_PER_TILE
    pltpu.sync_copy(scores_hbm, scores_v)
    pltpu.sync_copy(x_hbm.at[pl.ds(base * _SEQ, _ROWS_PER_TILE * _SEQ)], x_v)
    lane = lax.iota(jnp.int32, _LANES)
    zero = jnp.zeros((_LANES,), jnp.float32)
    for g in range(_ROWS_PER_TILE // _LANES):
        base_vec = (lane + g * _LANES) * _SEQ

        def body(jj, accs, base_vec=base_vec):
            a0, a1 = accs
            j0 = jj * _UNROLL
            for u in range(_UNROLL):
                idx = plsc.load_gather(x_v, [base_vec + (j0 + u)])
                val = plsc.load_gather(scores_v, [idx])
                if u % 2 == 0:
                    a0 = a0 + val
                else:
                    a1 = a1 + val
            return (a0, a1)

        a0, a1 = lax.fori_loop(0, _SEQ // _UNROLL, body, (zero, zero))
        out_v[pl.ds(g * _LANES, _LANES)] = a0 + a1
    pltpu.sync_copy(out_v, out_hbm.at[pl.ds(base, _ROWS_PER_TILE)])


def kernel(x, table, W, b):
    scores = _scores_call(table, W, b).reshape(_VOCAB)
    out = _sc_pool(scores, x.astype(jnp.int32).reshape(_BATCH * _SEQ))
    return out.reshape(_BATCH, 1)


# speedup vs baseline: 25.5017x; 1.1709x over previous
import functools

import jax
import jax.numpy as jnp
from jax import lax
from jax.experimental import pallas as pl
from jax.experimental.pallas import tpu as pltpu
from jax.experimental.pallas import tpu_sc as plsc

_VOCAB = 100000
_DIM = 128
_BATCH = 4096
_SEQ = 200

_ROWS_BLK = 1024
_NBLK = -(-_VOCAB // _ROWS_BLK)
_VPAD = _NBLK * _ROWS_BLK

_NC = 2
_NS = 16
_NW = _NC * _NS
_ROWS_PER_TILE = _BATCH // _NW
_LANES = 16
_UNROLL = 8


def _scores_body(t_ref, w_ref, b_ref, o_ref):
    i = pl.program_id(0)
    t = t_ref[...].reshape(8, _DIM, _DIM)
    w = w_ref[...]
    s = jnp.sum(t * w, axis=-1) * (1.0 / _SEQ)
    binv = b_ref[0] * (1.0 / _SEQ)
    s = s + binv

    @pl.when(i == 0)
    def _():
        r = lax.broadcasted_iota(jnp.int32, (8, _DIM), 0)
        c = lax.broadcasted_iota(jnp.int32, (8, _DIM), 1)
        o_ref[...] = jnp.where((r == 0) & (c == 0), binv, s).reshape(1, 8, _DIM)

    @pl.when(i != 0)
    def _():
        o_ref[...] = s.reshape(1, 8, _DIM)


_scores_call = pl.pallas_call(
    _scores_body,
    grid=(_NBLK,),
    in_specs=[
        pl.BlockSpec((_ROWS_BLK, _DIM), lambda i: (i, 0)),
        pl.BlockSpec((1, _DIM), lambda i: (0, 0)),
        pl.BlockSpec(memory_space=pltpu.SMEM),
    ],
    out_specs=pl.BlockSpec((1, 8, _DIM), lambda i: (i, 0, 0)),
    out_shape=jax.ShapeDtypeStruct((_NBLK, 8, _DIM), jnp.float32),
)


@functools.partial(
    pl.kernel,
    out_type=jax.ShapeDtypeStruct((_BATCH,), jnp.float32),
    mesh=plsc.VectorSubcoreMesh(core_axis_name="c", subcore_axis_name="s"),
    compiler_params=pltpu.CompilerParams(needs_layout_passes=False),
    scratch_types=[
        pltpu.VMEM((_VPAD,), jnp.float32),
        pltpu.VMEM((_ROWS_PER_TILE * _SEQ,), jnp.int32),
        pltpu.VMEM((_ROWS_PER_TILE,), jnp.float32),
    ],
)
def _sc_pool(scores_hbm, x_hbm, out_hbm, scores_v, x_v, out_v):
    wid = lax.axis_index("s") * _NC + lax.axis_index("c")
    base = wid * _ROWS_PER_TILE
    pltpu.sync_copy(scores_hbm, scores_v)
    pltpu.sync_copy(x_hbm.at[pl.ds(base * _SEQ, _ROWS_PER_TILE * _SEQ)], x_v)
    lane = lax.iota(jnp.int32, _LANES)
    zero = jnp.zeros((_LANES,), jnp.float32)
    for g in range(_ROWS_PER_TILE // _LANES):
        base_vec = (lane + g * _LANES) * _SEQ

        def body(jj, accs, base_vec=base_vec):
            a0, a1 = accs
            j0 = jj * _UNROLL
            for u in range(_UNROLL):
                idx = plsc.load_gather(x_v, [base_vec + (j0 + u)])
                val = plsc.load_gather(scores_v, [idx])
                if u % 2 == 0:
                    a0 = a0 + val
                else:
                    a1 = a1 + val
            return (a0, a1)

        a0, a1 = lax.fori_loop(0, _SEQ // _UNROLL, body, (zero, zero))
        out_v[pl.ds(g * _LANES, _LANES)] = a0 + a1
    pltpu.sync_copy(out_v, out_hbm.at[pl.ds(base, _ROWS_PER_TILE)])


def kernel(x, table, W, b):
    scores = _scores_call(table, W, b).reshape(_VPAD)
    out = _sc_pool(scores, x.astype(jnp.int32).reshape(_BATCH * _SEQ))
    return out.reshape(_BATCH, 1)


# speedup vs baseline: 26.2364x; 1.0288x over previous
import functools

import jax
import jax.numpy as jnp
from jax import lax
from jax.experimental import pallas as pl
from jax.experimental.pallas import tpu as pltpu
from jax.experimental.pallas import tpu_sc as plsc

_VOCAB = 100000
_DIM = 128
_BATCH = 4096
_SEQ = 200

_ROWS_BLK = 1024
_NBLK = -(-_VOCAB // _ROWS_BLK)
_VPAD = _NBLK * _ROWS_BLK

_NC = 2
_NS = 16
_NW = _NC * _NS
_ROWS_PER_TILE = _BATCH // _NW
_LANES = 16
_UNROLL = 8


def _scores_body(t_ref, w_ref, b_ref, o_ref):
    t = t_ref[...]
    w = w_ref[...]
    s = lax.dot_general(w, t, (((1,), (1,)), ((), ())),
                        preferred_element_type=jnp.float32)
    o_ref[...] = s.reshape(1, 8, _DIM) + b_ref[0] * (1.0 / _SEQ)


_scores_call = pl.pallas_call(
    _scores_body,
    grid=(_NBLK,),
    in_specs=[
        pl.BlockSpec((_ROWS_BLK, _DIM), lambda i: (i, 0)),
        pl.BlockSpec((1, _DIM), lambda i: (0, 0)),
        pl.BlockSpec(memory_space=pltpu.SMEM),
    ],
    out_specs=pl.BlockSpec((1, 8, _DIM), lambda i: (i, 0, 0)),
    out_shape=jax.ShapeDtypeStruct((_NBLK, 8, _DIM), jnp.float32),
)


@functools.partial(
    pl.kernel,
    out_type=jax.ShapeDtypeStruct((_BATCH,), jnp.float32),
    mesh=plsc.VectorSubcoreMesh(core_axis_name="c", subcore_axis_name="s"),
    compiler_params=pltpu.CompilerParams(needs_layout_passes=False),
    scratch_types=[
        pltpu.VMEM((_NBLK, 8, _DIM), jnp.float32),
        pltpu.VMEM((_ROWS_PER_TILE * _SEQ,), jnp.int32),
        pltpu.VMEM((_ROWS_PER_TILE,), jnp.float32),
    ],
)
def _sc_pool(scores_hbm, x_hbm, out_hbm, scores_v, x_v, out_v):
    wid = lax.axis_index("s") * _NC + lax.axis_index("c")
    base = wid * _ROWS_PER_TILE
    pltpu.sync_copy(scores_hbm, scores_v)
    pltpu.sync_copy(x_hbm.at[pl.ds(base * _SEQ, _ROWS_PER_TILE * _SEQ)], x_v)
    lane = lax.iota(jnp.int32, _LANES)
    zero = jnp.zeros((_LANES,), jnp.float32)
    for g in range(_ROWS_PER_TILE // _LANES):
        base_vec = (lane + g * _LANES) * _SEQ

        def body(jj, accs, base_vec=base_vec):
            a0, a1 = accs
            j0 = jj * _UNROLL
            for u in range(_UNROLL):
                idx = plsc.load_gather(x_v, [base_vec + (j0 + u)])
                val = plsc.load_gather(
                    scores_v,
                    [lax.shift_right_logical(idx, 10),
                     lax.shift_right_logical(idx, 7) & 7,
                     idx & 127])
                if u % 2 == 0:
                    a0 = a0 + val
                else:
                    a1 = a1 + val
            return (a0, a1)

        a0, a1 = lax.fori_loop(0, _SEQ // _UNROLL, body, (zero, zero))
        out_v[pl.ds(g * _LANES, _LANES)] = a0 + a1
    pltpu.sync_copy(out_v, out_hbm.at[pl.ds(base, _ROWS_PER_TILE)])


def kernel(x, table, W, b):
    scores = _scores_call(table, W * (1.0 / _SEQ), b)
    out = _sc_pool(scores, x.astype(jnp.int32).reshape(_BATCH * _SEQ))
    return out.reshape(_BATCH, 1)


# speedup vs baseline: 34.1201x; 1.3005x over previous
import functools

import jax
import jax.numpy as jnp
from jax import lax
from jax.experimental import pallas as pl
from jax.experimental.pallas import tpu as pltpu
from jax.experimental.pallas import tpu_sc as plsc

_VOCAB = 100000
_DIM = 128
_BATCH = 4096
_SEQ = 200

_ROWS_BLK = 2048
_NBLK = -(-_VOCAB // _ROWS_BLK)
_VPAD = _NBLK * _ROWS_BLK
_SUB = _ROWS_BLK // _DIM

_NC = 2
_NS = 16
_NW = _NC * _NS
_ROWS_PER_TILE = _BATCH // _NW
_LANES = 16
_UNROLL = 8


def _scores_body(t_ref, w_ref, b_ref, o_ref):
    t = t_ref[...]
    w = w_ref[...]
    s = lax.dot_general(w, t, (((1,), (1,)), ((), ())),
                        preferred_element_type=jnp.float32)
    o_ref[...] = s.reshape(1, _SUB, _DIM) + b_ref[0] * (1.0 / _SEQ)


_scores_call = pl.pallas_call(
    _scores_body,
    grid=(_NBLK,),
    in_specs=[
        pl.BlockSpec((_ROWS_BLK, _DIM), lambda i: (i, 0)),
        pl.BlockSpec((1, _DIM), lambda i: (0, 0)),
        pl.BlockSpec(memory_space=pltpu.SMEM),
    ],
    out_specs=pl.BlockSpec((1, _SUB, _DIM), lambda i: (i, 0, 0)),
    out_shape=jax.ShapeDtypeStruct((_NBLK, _SUB, _DIM), jnp.float32),
)


@functools.partial(
    pl.kernel,
    out_type=jax.ShapeDtypeStruct((_BATCH,), jnp.float32),
    mesh=plsc.VectorSubcoreMesh(core_axis_name="c", subcore_axis_name="s"),
    compiler_params=pltpu.CompilerParams(needs_layout_passes=False),
    scratch_types=[
        pltpu.VMEM((_NBLK, _SUB, _DIM), jnp.float32),
        pltpu.VMEM((_ROWS_PER_TILE * _SEQ,), jnp.int32),
        pltpu.VMEM((_ROWS_PER_TILE,), jnp.float32),
    ],
)
def _sc_pool(scores_hbm, x_hbm, out_hbm, scores_v, x_v, out_v):
    wid = lax.axis_index("s") * _NC + lax.axis_index("c")
    base = wid * _ROWS_PER_TILE
    pltpu.sync_copy(scores_hbm, scores_v)
    pltpu.sync_copy(x_hbm.at[pl.ds(base * _SEQ, _ROWS_PER_TILE * _SEQ)], x_v)
    lane = lax.iota(jnp.int32, _LANES)
    zero = jnp.zeros((_LANES,), jnp.float32)
    for g in range(_ROWS_PER_TILE // _LANES):
        base_vec = (lane + g * _LANES) * _SEQ

        def body(jj, accs, base_vec=base_vec):
            a0, a1 = accs
            j0 = jj * _UNROLL
            for u in range(_UNROLL):
                idx = plsc.load_gather(x_v, [base_vec + (j0 + u)])
                val = plsc.load_gather(
                    scores_v,
                    [lax.shift_right_logical(idx, 11),
                     lax.shift_right_logical(idx, 7) & (_SUB - 1),
                     idx & 127])
                if u % 2 == 0:
                    a0 = a0 + val
                else:
                    a1 = a1 + val
            return (a0, a1)

        a0, a1 = lax.fori_loop(0, _SEQ // _UNROLL, body, (zero, zero))
        out_v[pl.ds(g * _LANES, _LANES)] = a0 + a1
    pltpu.sync_copy(out_v, out_hbm.at[pl.ds(base, _ROWS_PER_TILE)])


def kernel(x, table, W, b):
    scores = _scores_call(table, W * (1.0 / _SEQ), b)
    out = _sc_pool(scores, x.astype(jnp.int32).reshape(_BATCH * _SEQ))
    return out.reshape(_BATCH, 1)


# speedup vs baseline: 39.5056x; 1.1578x over previous
import functools

import jax
import jax.numpy as jnp
from jax import lax
from jax.experimental import pallas as pl
from jax.experimental.pallas import tpu as pltpu
from jax.experimental.pallas import tpu_sc as plsc

_VOCAB = 100000
_DIM = 128
_BATCH = 4096
_SEQ = 200

_ROWS_BLK = 4096
_NBLK = -(-_VOCAB // _ROWS_BLK)
_VPAD = _NBLK * _ROWS_BLK
_SUB = _ROWS_BLK // _DIM

_NC = 2
_NS = 16
_NW = _NC * _NS
_ROWS_PER_TILE = _BATCH // _NW
_LANES = 16
_UNROLL = 8


def _scores_body(t_ref, w_ref, b_ref, o_ref):
    t = t_ref[...]
    w = w_ref[...]
    s = lax.dot_general(w, t, (((1,), (1,)), ((), ())),
                        preferred_element_type=jnp.float32)
    o_ref[...] = s.reshape(1, _SUB, _DIM) + b_ref[0] * (1.0 / _SEQ)


_scores_call = pl.pallas_call(
    _scores_body,
    grid=(_NBLK,),
    in_specs=[
        pl.BlockSpec((_ROWS_BLK, _DIM), lambda i: (i, 0)),
        pl.BlockSpec((1, _DIM), lambda i: (0, 0)),
        pl.BlockSpec(memory_space=pltpu.SMEM),
    ],
    out_specs=pl.BlockSpec((1, _SUB, _DIM), lambda i: (i, 0, 0)),
    out_shape=jax.ShapeDtypeStruct((_NBLK, _SUB, _DIM), jnp.float32),
)


@functools.partial(
    pl.kernel,
    out_type=jax.ShapeDtypeStruct((_BATCH,), jnp.float32),
    mesh=plsc.VectorSubcoreMesh(core_axis_name="c", subcore_axis_name="s"),
    compiler_params=pltpu.CompilerParams(needs_layout_passes=False),
    scratch_types=[
        pltpu.VMEM((_NBLK, _SUB, _DIM), jnp.float32),
        pltpu.VMEM((_ROWS_PER_TILE * _SEQ,), jnp.int32),
        pltpu.VMEM((_ROWS_PER_TILE,), jnp.float32),
    ],
)
def _sc_pool(scores_hbm, x_hbm, out_hbm, scores_v, x_v, out_v):
    wid = lax.axis_index("s") * _NC + lax.axis_index("c")
    base = wid * _ROWS_PER_TILE
    pltpu.sync_copy(scores_hbm, scores_v)
    pltpu.sync_copy(x_hbm.at[pl.ds(base * _SEQ, _ROWS_PER_TILE * _SEQ)], x_v)
    lane = lax.iota(jnp.int32, _LANES)
    zero = jnp.zeros((_LANES,), jnp.float32)
    for g in range(_ROWS_PER_TILE // _LANES):
        base_vec = (lane + g * _LANES) * _SEQ

        def body(jj, accs, base_vec=base_vec):
            a0, a1 = accs
            j0 = jj * _UNROLL
            for u in range(_UNROLL):
                idx = plsc.load_gather(x_v, [base_vec + (j0 + u)])
                val = plsc.load_gather(
                    scores_v,
                    [lax.shift_right_logical(idx, 12),
                     lax.shift_right_logical(idx, 7) & (_SUB - 1),
                     idx & 127])
                if u % 2 == 0:
                    a0 = a0 + val
                else:
                    a1 = a1 + val
            return (a0, a1)

        a0, a1 = lax.fori_loop(0, _SEQ // _UNROLL, body, (zero, zero))
        out_v[pl.ds(g * _LANES, _LANES)] = a0 + a1
    pltpu.sync_copy(out_v, out_hbm.at[pl.ds(base, _ROWS_PER_TILE)])


def kernel(x, table, W, b):
    scores = _scores_call(table, W * (1.0 / _SEQ), b)
    out = _sc_pool(scores, x.astype(jnp.int32).reshape(_BATCH * _SEQ))
    return out.reshape(_BATCH, 1)
